# Initial kernel scaffold; baseline (speedup 1.0000x reference)
import functools

import jax
import jax.numpy as jnp
from jax import lax
from jax.experimental import pallas as pl
from jax.experimental.pallas import tpu as pltpu
from jax.experimental.pallas import tpu_sc as plsc

N = 100000
E = 3200000
NPAD = 100096
NW = 32
CHUNK = 128
CPT = 782
EPAD = NW * CPT * CHUNK
GROUP = 23
SL = NPAD // 16

_mesh = plsc.VectorSubcoreMesh(core_axis_name="c", subcore_axis_name="s")


@functools.partial(
    pl.kernel,
    out_type=jax.ShapeDtypeStruct((2, NPAD), jnp.float32),
    mesh=_mesh,
    scratch_types=[
        pltpu.VMEM_SHARED((NPAD,), jnp.float32),
        pltpu.VMEM((GROUP, CHUNK), jnp.int32),
        pltpu.VMEM((CHUNK,), jnp.float32),
        pltpu.VMEM((SL,), jnp.float32),
    ],
)
def _deg_kernel(dst_hbm, zeros_hbm, degp_hbm, deg_sh, idxb, ones, zbuf):
    cid = lax.axis_index("c")
    sid = lax.axis_index("s")
    wid = cid * 16 + sid
    for i in range(CHUNK // 16):
        ones[pl.ds(i * 16, 16)] = jnp.ones((16,), jnp.float32)
    pltpu.sync_copy(zeros_hbm, zbuf)
    pltpu.sync_copy(zbuf, deg_sh.at[pl.ds(sid * SL, SL)])
    plsc.subcore_barrier()

    def outer(g, carry):
        pltpu.sync_copy(dst_hbm.at[wid, pl.ds(g * GROUP, GROUP)], idxb)
        for j in range(GROUP):
            pltpu.sync_copy(ones, deg_sh.at[idxb.at[j]], add=True)
        return carry

    lax.fori_loop(0, CPT // GROUP, outer, 0)
    plsc.subcore_barrier()
    pltpu.sync_copy(deg_sh.at[pl.ds(sid * SL, SL)], zbuf)
    pltpu.sync_copy(zbuf, degp_hbm.at[cid, pl.ds(sid * SL, SL)])


@functools.partial(
    pl.kernel,
    out_type=jax.ShapeDtypeStruct((2, NPAD, 4), jnp.float32),
    mesh=_mesh,
    scratch_types=[
        pltpu.VMEM_SHARED((NPAD, 4), jnp.float32),
        pltpu.VMEM((GROUP, CHUNK), jnp.int32),
        pltpu.VMEM((GROUP, CHUNK), jnp.int32),
        pltpu.VMEM((GROUP, CHUNK, 4), jnp.float32),
        pltpu.VMEM((SL, 4), jnp.float32),
        pltpu.SemaphoreType.DMA,
    ],
)
def _agg_kernel(src_hbm, dst_hbm, yd_hbm, zeros4_hbm, accp_hbm,
                acc_sh, sbuf, dbuf, rows, zbuf, sem):
    cid = lax.axis_index("c")
    sid = lax.axis_index("s")
    wid = cid * 16 + sid
    pltpu.sync_copy(zeros4_hbm, zbuf)
    pltpu.sync_copy(zbuf, acc_sh.at[pl.ds(sid * SL, SL)])
    plsc.subcore_barrier()

    def outer(g, carry):
        pltpu.sync_copy(src_hbm.at[wid, pl.ds(g * GROUP, GROUP)], sbuf)
        pltpu.sync_copy(dst_hbm.at[wid, pl.ds(g * GROUP, GROUP)], dbuf)
        copies = [
            pltpu.async_copy(yd_hbm.at[sbuf.at[j]], rows.at[j], sem)
            for j in range(GROUP)
        ]
        for c in copies:
            c.wait()
        for j in range(GROUP):
            pltpu.sync_copy(rows.at[j], acc_sh.at[dbuf.at[j]], add=True)
        return carry

    lax.fori_loop(0, CPT // GROUP, outer, 0)
    plsc.subcore_barrier()
    pltpu.sync_copy(acc_sh.at[pl.ds(sid * SL, SL)], zbuf)
    pltpu.sync_copy(zbuf, accp_hbm.at[cid, pl.ds(sid * SL, SL)])


_R = NPAD // 16


def _dense_body(x_ref, dg_ref, w_ref, yd_ref):
    deg = dg_ref[:, 0:1] + dg_ref[:, 1:2] + 1.0
    dinv = lax.rsqrt(deg)
    x = x_ref[...]
    cols = []
    for j in range(3):
        c = (x[:, 0:1] * w_ref[j, 0] + x[:, 1:2] * w_ref[j, 1]
             + x[:, 2:3] * w_ref[j, 2])
        cols.append(c * dinv)
    cols.append(dinv)
    yd_ref[...] = jnp.concatenate(cols, axis=1)


def _comb_body(acc_ref, yd_ref, b_ref, out_ref):
    a = acc_ref[...]
    agg = a[0] + a[1]
    yd = yd_ref[...]
    dinv = yd[:, 3:4]
    out_ref[...] = (agg + yd) * dinv + b_ref[...]


def kernel(x, edge_index, W, b):
    ei = edge_index.astype(jnp.int32)
    pad = jnp.full((2, EPAD - E), N, jnp.int32)
    ei = jnp.concatenate([ei, pad], axis=1)
    src3 = ei[0].reshape(NW, CPT, CHUNK)
    dst3 = ei[1].reshape(NW, CPT, CHUNK)

    zeros1 = jnp.zeros((SL,), jnp.float32)
    zeros4 = jnp.zeros((SL, 4), jnp.float32)
    degp = _deg_kernel(dst3, zeros1)

    x4 = jnp.pad(x, ((0, NPAD - N), (0, 1)))
    degT = degp.T
    yd = pl.pallas_call(
        _dense_body,
        grid=(16,),
        in_specs=[
            pl.BlockSpec((_R, 4), lambda i: (i, 0)),
            pl.BlockSpec((_R, 2), lambda i: (i, 0)),
            pl.BlockSpec(memory_space=pltpu.SMEM),
        ],
        out_specs=pl.BlockSpec((_R, 4), lambda i: (i, 0)),
        out_shape=jax.ShapeDtypeStruct((NPAD, 4), jnp.float32),
    )(x4, degT, W)

    accp = _agg_kernel(src3, dst3, yd, zeros4)

    bp = jnp.pad(b, (0, 1)).reshape(1, 4)
    out = pl.pallas_call(
        _comb_body,
        grid=(16,),
        in_specs=[
            pl.BlockSpec((2, _R, 4), lambda i: (0, i, 0)),
            pl.BlockSpec((_R, 4), lambda i: (i, 0)),
            pl.BlockSpec((1, 4), lambda i: (0, 0)),
        ],
        out_specs=pl.BlockSpec((_R, 4), lambda i: (i, 0)),
        out_shape=jax.ShapeDtypeStruct((NPAD, 4), jnp.float32),
    )(accp, yd, bp)
    return out[:N, :3]


# speedup vs baseline: 77.2850x; 77.2850x over previous
import functools

import jax
import jax.numpy as jnp
from jax import lax
from jax.experimental import pallas as pl
from jax.experimental.pallas import tpu as pltpu
from jax.experimental.pallas import tpu_sc as plsc

N = 100000
E = 3200000
NPAD = 100096
NW = 32
CHUNK = 128
CPT = 784
EPAD = NW * CPT * CHUNK
GROUP = 16
AGROUP = 8
SL = NPAD // 16
PIECE = 368

_mesh = plsc.VectorSubcoreMesh(core_axis_name="c", subcore_axis_name="s")
_sc_params = pltpu.CompilerParams(use_tc_tiling_on_sc=False)


@functools.partial(
    pl.kernel,
    out_type=jax.ShapeDtypeStruct((2 * NPAD,), jnp.float32),
    mesh=_mesh,
    compiler_params=_sc_params,
    scratch_types=[
        pltpu.VMEM_SHARED((NPAD,), jnp.float32),
        pltpu.VMEM((GROUP, CHUNK), jnp.int32),
        pltpu.VMEM((CHUNK,), jnp.float32),
        pltpu.VMEM((SL,), jnp.float32),
    ],
)
def _deg_kernel(dst_hbm, zeros_hbm, degp_hbm, deg_sh, idxb, ones, zbuf):
    cid = lax.axis_index("c")
    sid = lax.axis_index("s")
    wid = cid * 16 + sid
    for i in range(CHUNK // 16):
        ones[pl.ds(i * 16, 16)] = jnp.ones((16,), jnp.float32)
    pltpu.sync_copy(zeros_hbm, zbuf)
    pltpu.sync_copy(zbuf, deg_sh.at[pl.ds(sid * SL, SL)])
    plsc.subcore_barrier()

    def outer(g, carry):
        pltpu.sync_copy(dst_hbm.at[wid, pl.ds(g * GROUP, GROUP)], idxb)
        for j in range(GROUP):
            pltpu.sync_copy(ones, deg_sh.at[idxb.at[j]], add=True)
        return carry

    lax.fori_loop(0, CPT // GROUP, outer, 0)
    plsc.subcore_barrier()
    pltpu.sync_copy(deg_sh.at[pl.ds(sid * SL, SL)], zbuf)
    pltpu.sync_copy(zbuf, degp_hbm.at[pl.ds(cid * NPAD + sid * SL, SL)])


@functools.partial(
    pl.kernel,
    out_type=jax.ShapeDtypeStruct((2, NPAD, 8), jnp.float32),
    mesh=_mesh,
    compiler_params=_sc_params,
    scratch_types=[
        pltpu.VMEM_SHARED((NPAD, 8), jnp.float32),
        pltpu.VMEM_SHARED((NPAD, 8), jnp.float32),
        pltpu.VMEM((AGROUP, CHUNK), jnp.int32),
        pltpu.VMEM((AGROUP, CHUNK), jnp.int32),
        pltpu.VMEM((AGROUP, CHUNK, 8), jnp.float32),
        pltpu.VMEM((PIECE, 8), jnp.float32),
        pltpu.SemaphoreType.DMA,
    ],
)
def _agg_kernel(src_hbm, dst_hbm, yd_hbm, zeros4_hbm, accp_hbm,
                acc_sh, y_sh, sbuf, dbuf, rows, zbuf, sem):
    cid = lax.axis_index("c")
    sid = lax.axis_index("s")
    wid = cid * 16 + sid
    pltpu.sync_copy(zeros4_hbm, zbuf)

    def prep(t, carry):
        off = sid * SL + t * PIECE
        pltpu.sync_copy(zbuf, acc_sh.at[pl.ds(off, PIECE)])
        return carry

    lax.fori_loop(0, SL // PIECE, prep, 0)

    def stage(t, carry):
        off = sid * SL + t * PIECE
        pltpu.sync_copy(yd_hbm.at[pl.ds(off, PIECE)], zbuf)
        pltpu.sync_copy(zbuf, y_sh.at[pl.ds(off, PIECE)])
        return carry

    lax.fori_loop(0, SL // PIECE, stage, 0)
    plsc.subcore_barrier()

    def outer(g, carry):
        pltpu.sync_copy(src_hbm.at[wid, pl.ds(g * AGROUP, AGROUP)], sbuf)
        pltpu.sync_copy(dst_hbm.at[wid, pl.ds(g * AGROUP, AGROUP)], dbuf)
        copies = [
            pltpu.async_copy(y_sh.at[sbuf.at[j]], rows.at[j], sem)
            for j in range(AGROUP)
        ]
        for c in copies:
            c.wait()
        for j in range(AGROUP):
            pltpu.sync_copy(rows.at[j], acc_sh.at[dbuf.at[j]], add=True)
        return carry

    lax.fori_loop(0, CPT // AGROUP, outer, 0)
    plsc.subcore_barrier()

    def copyout(t, carry):
        off = sid * SL + t * PIECE
        pltpu.sync_copy(acc_sh.at[pl.ds(off, PIECE)], zbuf)
        pltpu.sync_copy(zbuf, accp_hbm.at[cid, pl.ds(off, PIECE)])
        return carry

    lax.fori_loop(0, SL // PIECE, copyout, 0)


_R = NPAD // 16


def _dense_body(x_ref, dg_ref, w_ref, yd_ref):
    deg = dg_ref[:, 0:1] + dg_ref[:, 1:2] + 1.0
    dinv = lax.rsqrt(deg)
    x = x_ref[...]
    cols = []
    for j in range(3):
        c = (x[:, 0:1] * w_ref[j, 0] + x[:, 1:2] * w_ref[j, 1]
             + x[:, 2:3] * w_ref[j, 2])
        cols.append(c * dinv)
    cols.append(dinv)
    z = dinv * 0.0
    cols.extend([z, z, z, z])
    yd_ref[...] = jnp.concatenate(cols, axis=1)


def _comb_body(acc_ref, yd_ref, b_ref, out_ref):
    a = acc_ref[...]
    agg = a[0] + a[1]
    yd = yd_ref[...]
    dinv = yd[:, 3:4]
    out_ref[...] = (agg + yd) * dinv + b_ref[...]


def kernel(x, edge_index, W, b):
    ei = edge_index.astype(jnp.int32)
    pad = jnp.full((2, EPAD - E), N, jnp.int32)
    ei = jnp.concatenate([ei, pad], axis=1)
    src3 = ei[0].reshape(NW, CPT, CHUNK)
    dst3 = ei[1].reshape(NW, CPT, CHUNK)

    zeros1 = jnp.zeros((SL,), jnp.float32)
    zeros4 = jnp.zeros((PIECE, 8), jnp.float32)
    degp = _deg_kernel(dst3, zeros1).reshape(2, NPAD)

    x4 = jnp.pad(x, ((0, NPAD - N), (0, 1)))
    degT = degp.T
    yd = pl.pallas_call(
        _dense_body,
        grid=(16,),
        in_specs=[
            pl.BlockSpec((_R, 4), lambda i: (i, 0)),
            pl.BlockSpec((_R, 2), lambda i: (i, 0)),
            pl.BlockSpec(memory_space=pltpu.SMEM),
        ],
        out_specs=pl.BlockSpec((_R, 8), lambda i: (i, 0)),
        out_shape=jax.ShapeDtypeStruct((NPAD, 8), jnp.float32),
    )(x4, degT, W)

    accp = _agg_kernel(src3, dst3, yd, zeros4)

    bp = jnp.pad(b, (0, 5)).reshape(1, 8)
    out = pl.pallas_call(
        _comb_body,
        grid=(16,),
        in_specs=[
            pl.BlockSpec((2, _R, 8), lambda i: (0, i, 0)),
            pl.BlockSpec((_R, 8), lambda i: (i, 0)),
            pl.BlockSpec((1, 8), lambda i: (0, 0)),
        ],
        out_specs=pl.BlockSpec((_R, 8), lambda i: (i, 0)),
        out_shape=jax.ShapeDtypeStruct((NPAD, 8), jnp.float32),
    )(accp, yd, bp)
    return out[:N, :3]


# speedup vs baseline: 88.1868x; 1.1411x over previous
import functools

import jax
import jax.numpy as jnp
from jax import lax
from jax.experimental import pallas as pl
from jax.experimental.pallas import tpu as pltpu
from jax.experimental.pallas import tpu_sc as plsc

N = 100000
E = 3200000
NPAD = 100096
NW = 32
CHUNK = 128
CPT = 784
EPAD = NW * CPT * CHUNK
GROUP = 28
AGROUP = 8
SL = NPAD // 16
PIECE = 368

_mesh = plsc.VectorSubcoreMesh(core_axis_name="c", subcore_axis_name="s")
_sc_params = pltpu.CompilerParams(use_tc_tiling_on_sc=False)


@functools.partial(
    pl.kernel,
    out_type=jax.ShapeDtypeStruct((2 * NPAD,), jnp.float32),
    mesh=_mesh,
    compiler_params=_sc_params,
    scratch_types=[
        pltpu.VMEM_SHARED((NPAD,), jnp.float32),
        pltpu.VMEM((GROUP, CHUNK), jnp.int32),
        pltpu.VMEM((GROUP, CHUNK), jnp.int32),
        pltpu.VMEM((CHUNK,), jnp.float32),
        pltpu.VMEM((SL,), jnp.float32),
        pltpu.SemaphoreType.DMA,
    ],
)
def _deg_kernel(dst_hbm, zeros_hbm, degp_hbm, deg_sh, idxa, idxb, ones, zbuf,
                ssem):
    cid = lax.axis_index("c")
    sid = lax.axis_index("s")
    wid = cid * 16 + sid
    for i in range(CHUNK // 16):
        ones[pl.ds(i * 16, 16)] = jnp.ones((16,), jnp.float32)
    pltpu.sync_copy(zeros_hbm, zbuf)
    pltpu.sync_copy(zbuf, deg_sh.at[pl.ds(sid * SL, SL)])
    plsc.subcore_barrier()

    def outer(i, carry):
        ga = 2 * i
        gb = 2 * i + 1
        pltpu.sync_copy(dst_hbm.at[wid, pl.ds(ga * GROUP, GROUP)], idxa)
        ca = [pltpu.async_copy(ones, deg_sh.at[idxa.at[j]], ssem, add=True)
              for j in range(GROUP)]
        pltpu.sync_copy(dst_hbm.at[wid, pl.ds(gb * GROUP, GROUP)], idxb)
        for c in ca:
            c.wait()
        cb = [pltpu.async_copy(ones, deg_sh.at[idxb.at[j]], ssem, add=True)
              for j in range(GROUP)]
        for c in cb:
            c.wait()
        return carry

    lax.fori_loop(0, CPT // GROUP // 2, outer, 0)
    plsc.subcore_barrier()
    pltpu.sync_copy(deg_sh.at[pl.ds(sid * SL, SL)], zbuf)
    pltpu.sync_copy(zbuf, degp_hbm.at[pl.ds(cid * NPAD + sid * SL, SL)])


@functools.partial(
    pl.kernel,
    out_type=jax.ShapeDtypeStruct((2, NPAD, 8), jnp.float32),
    mesh=_mesh,
    compiler_params=_sc_params,
    scratch_types=[
        pltpu.VMEM_SHARED((NPAD, 8), jnp.float32),
        pltpu.VMEM_SHARED((NPAD, 8), jnp.float32),
        pltpu.VMEM((AGROUP, CHUNK), jnp.int32),
        pltpu.VMEM((AGROUP, CHUNK), jnp.int32),
        pltpu.VMEM((AGROUP, CHUNK), jnp.int32),
        pltpu.VMEM((AGROUP, CHUNK), jnp.int32),
        pltpu.VMEM((AGROUP, CHUNK, 8), jnp.float32),
        pltpu.VMEM((AGROUP, CHUNK, 8), jnp.float32),
        pltpu.VMEM((PIECE, 8), jnp.float32),
        pltpu.SemaphoreType.DMA,
        pltpu.SemaphoreType.DMA,
    ],
)
def _agg_kernel(src_hbm, dst_hbm, yd_hbm, zeros4_hbm, accp_hbm,
                acc_sh, y_sh, sbufa, dbufa, sbufb, dbufb, rowsa, rowsb, zbuf,
                gsem, ssem):
    cid = lax.axis_index("c")
    sid = lax.axis_index("s")
    wid = cid * 16 + sid
    pltpu.sync_copy(zeros4_hbm, zbuf)

    def prep(t, carry):
        off = sid * SL + t * PIECE
        pltpu.sync_copy(zbuf, acc_sh.at[pl.ds(off, PIECE)])
        return carry

    lax.fori_loop(0, SL // PIECE, prep, 0)

    def stage(t, carry):
        off = sid * SL + t * PIECE
        pltpu.sync_copy(yd_hbm.at[pl.ds(off, PIECE)], zbuf)
        pltpu.sync_copy(zbuf, y_sh.at[pl.ds(off, PIECE)])
        return carry

    lax.fori_loop(0, SL // PIECE, stage, 0)
    plsc.subcore_barrier()

    def outer(i, carry):
        ga = 2 * i
        gb = 2 * i + 1
        pltpu.sync_copy(src_hbm.at[wid, pl.ds(ga * AGROUP, AGROUP)], sbufa)
        pltpu.sync_copy(dst_hbm.at[wid, pl.ds(ga * AGROUP, AGROUP)], dbufa)
        gath_a = [pltpu.async_copy(y_sh.at[sbufa.at[j]], rowsa.at[j], gsem)
                  for j in range(AGROUP)]
        pltpu.sync_copy(src_hbm.at[wid, pl.ds(gb * AGROUP, AGROUP)], sbufb)
        pltpu.sync_copy(dst_hbm.at[wid, pl.ds(gb * AGROUP, AGROUP)], dbufb)
        for c in gath_a:
            c.wait()
        scat_a = [pltpu.async_copy(rowsa.at[j], acc_sh.at[dbufa.at[j]], ssem,
                                   add=True)
                  for j in range(AGROUP)]
        gath_b = [pltpu.async_copy(y_sh.at[sbufb.at[j]], rowsb.at[j], gsem)
                  for j in range(AGROUP)]
        for c in gath_b:
            c.wait()
        for c in scat_a:
            c.wait()
        scat_b = [pltpu.async_copy(rowsb.at[j], acc_sh.at[dbufb.at[j]], ssem,
                                   add=True)
                  for j in range(AGROUP)]
        for c in scat_b:
            c.wait()
        return carry

    lax.fori_loop(0, CPT // AGROUP // 2, outer, 0)
    plsc.subcore_barrier()

    def copyout(t, carry):
        off = sid * SL + t * PIECE
        pltpu.sync_copy(acc_sh.at[pl.ds(off, PIECE)], zbuf)
        pltpu.sync_copy(zbuf, accp_hbm.at[cid, pl.ds(off, PIECE)])
        return carry

    lax.fori_loop(0, SL // PIECE, copyout, 0)


_R = NPAD // 16


def _dense_body(x_ref, dg_ref, w_ref, yd_ref):
    deg = dg_ref[...] + 1.0
    dinv = lax.rsqrt(deg)
    x = x_ref[...]
    cols = []
    for j in range(3):
        c = (x[:, 0:1] * w_ref[j, 0] + x[:, 1:2] * w_ref[j, 1]
             + x[:, 2:3] * w_ref[j, 2])
        cols.append(c * dinv)
    cols.append(dinv)
    z = dinv * 0.0
    cols.extend([z, z, z, z])
    yd_ref[...] = jnp.concatenate(cols, axis=1)


def _comb_body(acc_ref, yd_ref, b_ref, out_ref):
    a = acc_ref[...]
    agg = a[0] + a[1]
    yd = yd_ref[...]
    dinv = yd[:, 3:4]
    out_ref[...] = (agg + yd) * dinv + b_ref[...]


def kernel(x, edge_index, W, b):
    ei = edge_index.astype(jnp.int32)
    pad = jnp.full((2, EPAD - E), N, jnp.int32)
    ei = jnp.concatenate([ei, pad], axis=1)
    src3 = ei[0].reshape(NW, CPT, CHUNK)
    dst3 = ei[1].reshape(NW, CPT, CHUNK)

    zeros1 = jnp.zeros((SL,), jnp.float32)
    zeros4 = jnp.zeros((PIECE, 8), jnp.float32)
    degp = _deg_kernel(dst3, zeros1).reshape(2, NPAD)
    degs = (degp[0] + degp[1]).reshape(NPAD, 1)

    x4 = jnp.pad(x, ((0, NPAD - N), (0, 1)))
    yd = pl.pallas_call(
        _dense_body,
        grid=(16,),
        in_specs=[
            pl.BlockSpec((_R, 4), lambda i: (i, 0)),
            pl.BlockSpec((_R, 1), lambda i: (i, 0)),
            pl.BlockSpec(memory_space=pltpu.SMEM),
        ],
        out_specs=pl.BlockSpec((_R, 8), lambda i: (i, 0)),
        out_shape=jax.ShapeDtypeStruct((NPAD, 8), jnp.float32),
    )(x4, degs, W)

    accp = _agg_kernel(src3, dst3, yd, zeros4)

    bp = jnp.pad(b, (0, 5)).reshape(1, 8)
    out = pl.pallas_call(
        _comb_body,
        grid=(16,),
        in_specs=[
            pl.BlockSpec((2, _R, 8), lambda i: (0, i, 0)),
            pl.BlockSpec((_R, 8), lambda i: (i, 0)),
            pl.BlockSpec((1, 8), lambda i: (0, 0)),
        ],
        out_specs=pl.BlockSpec((_R, 8), lambda i: (i, 0)),
        out_shape=jax.ShapeDtypeStruct((NPAD, 8), jnp.float32),
    )(accp, yd, bp)
    return out[:N, :3]
